# baseline (speedup 1.0000x reference)
import jax
import jax.numpy as jnp
from jax import lax
from jax.experimental import pallas as pl
from jax.experimental.pallas import tpu as pltpu
from jax.experimental.pallas import tpu_sc as plsc

NUM_CORES = 2
NUM_SUBCORES = 16
LANES = 16
NW = NUM_CORES * NUM_SUBCORES

EMBED = 64
BATCH = 16384
ROWS_PER_W = BATCH // NW
CHUNK = 128
NCHUNK = ROWS_PER_W // CHUNK
KSUB = CHUNK // LANES


def _dot_kernel(uids_hbm, mids_hbm, utab_hbm, mtab_hbm, out_hbm,
                uidx_v, midx_v, ubuf_v, mbuf_v, out_v, usem, msem):
    wid = lax.axis_index("s") * NUM_CORES + lax.axis_index("c")
    base = wid * ROWS_PER_W

    pltpu.sync_copy(uids_hbm.at[wid], uidx_v)
    pltpu.sync_copy(mids_hbm.at[wid], midx_v)

    u_dsrc = utab_hbm.at[:, pl.ds(0, CHUNK)]
    m_dsrc = mtab_hbm.at[:, pl.ds(0, CHUNK)]

    @pl.loop(0, NCHUNK)
    def _(c):
        @pl.loop(0, EMBED)
        def _(j):
            pltpu.async_copy(utab_hbm.at[j].at[uidx_v.at[c]],
                             ubuf_v.at[j], usem)
            pltpu.async_copy(mtab_hbm.at[j].at[midx_v.at[c]],
                             mbuf_v.at[j], msem)

        pltpu.make_async_copy(u_dsrc, ubuf_v, usem).wait()
        pltpu.make_async_copy(m_dsrc, mbuf_v, msem).wait()

        acc = [jnp.zeros((LANES,), jnp.float32) for _ in range(KSUB)]
        for j in range(EMBED):
            for k in range(KSUB):
                sl = pl.ds(k * LANES, LANES)
                acc[k] = acc[k] + ubuf_v[j, sl] * mbuf_v[j, sl]
        for k in range(KSUB):
            out_v[pl.ds(c * CHUNK + k * LANES, LANES)] = acc[k]

    pltpu.sync_copy(out_v, out_hbm.at[pl.ds(base, ROWS_PER_W)])


@jax.jit
def _run(user_ids, movie_ids, user_table, movie_table):
    mesh = plsc.VectorSubcoreMesh(core_axis_name="c", subcore_axis_name="s",
                                  num_cores=NUM_CORES,
                                  num_subcores=NUM_SUBCORES)
    cp = pltpu.CompilerParams(needs_layout_passes=False,
                              use_tc_tiling_on_sc=False)
    kern = pl.kernel(
        _dot_kernel,
        out_type=jax.ShapeDtypeStruct((BATCH,), jnp.float32),
        mesh=mesh,
        compiler_params=cp,
        scratch_types=[
            pltpu.VMEM((NCHUNK, CHUNK), jnp.int32),
            pltpu.VMEM((NCHUNK, CHUNK), jnp.int32),
            pltpu.VMEM((EMBED, CHUNK), jnp.float32),
            pltpu.VMEM((EMBED, CHUNK), jnp.float32),
            pltpu.VMEM((ROWS_PER_W,), jnp.float32),
            pltpu.SemaphoreType.DMA,
            pltpu.SemaphoreType.DMA,
        ],
    )
    uids = user_ids.astype(jnp.int32).reshape(NW, NCHUNK, CHUNK)
    mids = movie_ids.astype(jnp.int32).reshape(NW, NCHUNK, CHUNK)
    return kern(uids, mids, user_table.T, movie_table.T)


def kernel(user_ids, movie_ids, user_table, movie_table):
    out = _run(user_ids, movie_ids, user_table, movie_table)
    return out.reshape(BATCH, 1)


# speedup vs baseline: 7.5027x; 7.5027x over previous
import jax
import jax.numpy as jnp
from jax import lax
from jax.experimental import pallas as pl
from jax.experimental.pallas import tpu as pltpu
from jax.experimental.pallas import tpu_sc as plsc

NUM_CORES = 2
NUM_SUBCORES = 16
LANES = 16
NW = NUM_CORES * NUM_SUBCORES

EMBED = 64
BATCH = 16384
ROWS_PER_W = BATCH // NW
CHUNK = 128
NCHUNK = ROWS_PER_W // CHUNK


def _dot_kernel(uids_hbm, mids_hbm, utab_hbm, mtab_hbm, out_hbm,
                uidx_v, midx_v, urows_v, mrows_v, out_v, sem):
    wid = lax.axis_index("s") * NUM_CORES + lax.axis_index("c")
    base = wid * ROWS_PER_W

    pltpu.sync_copy(uids_hbm.at[wid], uidx_v)
    pltpu.sync_copy(mids_hbm.at[wid], midx_v)

    copies = []
    for c in range(NCHUNK):
        sl = pl.ds(c * CHUNK, CHUNK)
        copies.append(pltpu.async_copy(utab_hbm.at[uidx_v.at[c]],
                                       urows_v.at[sl], sem))
        copies.append(pltpu.async_copy(mtab_hbm.at[midx_v.at[c]],
                                       mrows_v.at[sl], sem))
    for cp in copies:
        cp.wait()

    iota = lax.iota(jnp.int32, LANES)

    @pl.loop(0, ROWS_PER_W, step=LANES)
    def _(r0):
        rows = r0 + iota
        acc = jnp.zeros((LANES,), jnp.float32)
        for j in range(EMBED):
            col = jnp.full((LANES,), j, jnp.int32)
            u = plsc.load_gather(urows_v, [rows, col])
            m = plsc.load_gather(mrows_v, [rows, col])
            acc = acc + u * m
        out_v[pl.ds(r0, LANES)] = acc

    pltpu.sync_copy(out_v, out_hbm.at[pl.ds(base, ROWS_PER_W)])


@jax.jit
def _run(user_ids, movie_ids, user_table, movie_table):
    mesh = plsc.VectorSubcoreMesh(core_axis_name="c", subcore_axis_name="s",
                                  num_cores=NUM_CORES,
                                  num_subcores=NUM_SUBCORES)
    cp = pltpu.CompilerParams(needs_layout_passes=False,
                              use_tc_tiling_on_sc=False)
    kern = pl.kernel(
        _dot_kernel,
        compiler_params=cp,
        out_type=jax.ShapeDtypeStruct((BATCH,), jnp.float32),
        mesh=mesh,
        scratch_types=[
            pltpu.VMEM((NCHUNK, CHUNK), jnp.int32),
            pltpu.VMEM((NCHUNK, CHUNK), jnp.int32),
            pltpu.VMEM((ROWS_PER_W, EMBED), jnp.float32),
            pltpu.VMEM((ROWS_PER_W, EMBED), jnp.float32),
            pltpu.VMEM((ROWS_PER_W,), jnp.float32),
            pltpu.SemaphoreType.DMA,
        ],
    )
    uids = user_ids.astype(jnp.int32).reshape(NW, NCHUNK, CHUNK)
    mids = movie_ids.astype(jnp.int32).reshape(NW, NCHUNK, CHUNK)
    return kern(uids, mids, user_table, movie_table)


def kernel(user_ids, movie_ids, user_table, movie_table):
    out = _run(user_ids, movie_ids, user_table, movie_table)
    return out.reshape(BATCH, 1)
